# baseline (speedup 1.0000x reference)
import functools

import jax
import jax.numpy as jnp
from jax.experimental import pallas as pl
from jax.experimental.pallas import tpu as pltpu


def _body(J, T, C, sref, *refs):
    joints_ref = refs[T]
    out_ref = refs[T + 1]
    out_ref[:, :J, :] = joints_ref[...]
    for t in range(T):
        r = sref[t] % 8
        out_ref[:, J + t, :] = refs[t][:, pl.ds(r, 1), :][:, 0, :]


def _build_call(B, V, J, T, C, GB=512):
    def vert_spec(t):
        return pl.BlockSpec(
            (C, 8, GB), lambda i, sref, t=t: (0, sref[t] // 8, i))

    grid_spec = pltpu.PrefetchScalarGridSpec(
        num_scalar_prefetch=1,
        grid=(B // GB,),
        in_specs=(
            [vert_spec(t) for t in range(T)]
            + [pl.BlockSpec((C, J, GB), lambda i, sref: (0, 0, i))]
        ),
        out_specs=pl.BlockSpec((C, J + T, GB), lambda i, sref: (0, 0, i)),
    )
    return pl.pallas_call(
        functools.partial(_body, J, T, C),
        grid_spec=grid_spec,
        out_shape=jax.ShapeDtypeStruct((C, J + T, B), jnp.float32),
    )


def kernel(vertices, joints, extra_joints_idxs):
    B, V, C = vertices.shape
    J = joints.shape[1]
    T = extra_joints_idxs.shape[0]
    eidx = extra_joints_idxs.astype(jnp.int32)
    vt = vertices.transpose(2, 1, 0)
    jt = joints.transpose(2, 1, 0)
    out_t = _build_call(B, V, J, T, C)(eidx, *([vt] * T), jt)
    return out_t.transpose(2, 1, 0)


# speedup vs baseline: 1.1077x; 1.1077x over previous
import functools

import jax
import jax.numpy as jnp
from jax.experimental import pallas as pl
from jax.experimental.pallas import tpu as pltpu


def _body(J, T, C, sref, vert_ref, joints_ref, out_ref, *scratch):
    slivers = scratch[:T]
    sems = scratch[T]
    cps = [
        pltpu.make_async_copy(
            vert_ref.at[:, pl.ds(sref[t], 1), :], slivers[t], sems.at[t])
        for t in range(T)
    ]
    for c in cps:
        c.start()
    out_ref[:, :J, :] = joints_ref[...]
    for t in range(T):
        cps[t].wait()
        out_ref[:, J + t, :] = slivers[t][:, 0, :]


def _build_call(B, V, J, T, C):
    grid_spec = pltpu.PrefetchScalarGridSpec(
        num_scalar_prefetch=1,
        grid=(1,),
        in_specs=[
            pl.BlockSpec(memory_space=pl.ANY),
            pl.BlockSpec((C, J, B), lambda i, sref: (0, 0, 0)),
        ],
        out_specs=pl.BlockSpec((C, J + T, B), lambda i, sref: (0, 0, 0)),
        scratch_shapes=(
            [pltpu.VMEM((C, 1, B), jnp.float32) for _ in range(T)]
            + [pltpu.SemaphoreType.DMA((T,))]
        ),
    )
    return pl.pallas_call(
        functools.partial(_body, J, T, C),
        grid_spec=grid_spec,
        out_shape=jax.ShapeDtypeStruct((C, J + T, B), jnp.float32),
    )


def kernel(vertices, joints, extra_joints_idxs):
    B, V, C = vertices.shape
    J = joints.shape[1]
    T = extra_joints_idxs.shape[0]
    eidx = extra_joints_idxs.astype(jnp.int32)
    vt = vertices.transpose(2, 1, 0)
    jt = joints.transpose(2, 1, 0)
    out_t = _build_call(B, V, J, T, C)(eidx, vt, jt)
    return out_t.transpose(2, 1, 0)


# speedup vs baseline: 1.2679x; 1.1446x over previous
import functools

import jax
import jax.numpy as jnp
from jax.experimental import pallas as pl
from jax.experimental.pallas import tpu as pltpu


def _body(J, T, C, sref, *refs):
    joints_ref = refs[T]
    out_ref = refs[T + 1]
    out_ref[:, :J, :] = joints_ref[...]
    for t in range(T):
        r = sref[t] % 8
        out_ref[:, J + t, :] = refs[t][:, pl.ds(r, 1), :][:, 0, :]


def _build_call(B, V, J, T, C):
    def vert_spec(t):
        return pl.BlockSpec(
            (C, 8, B), lambda i, sref, t=t: (0, sref[t] // 8, 0))

    grid_spec = pltpu.PrefetchScalarGridSpec(
        num_scalar_prefetch=1,
        grid=(1,),
        in_specs=(
            [vert_spec(t) for t in range(T)]
            + [pl.BlockSpec((C, J, B), lambda i, sref: (0, 0, 0))]
        ),
        out_specs=pl.BlockSpec((C, J + T, B), lambda i, sref: (0, 0, 0)),
    )
    return pl.pallas_call(
        functools.partial(_body, J, T, C),
        grid_spec=grid_spec,
        out_shape=jax.ShapeDtypeStruct((C, J + T, B), jnp.float32),
    )


def kernel(vertices, joints, extra_joints_idxs):
    B, V, C = vertices.shape
    J = joints.shape[1]
    T = extra_joints_idxs.shape[0]
    eidx = extra_joints_idxs.astype(jnp.int32)
    vt = vertices.transpose(2, 1, 0)
    jt = joints.transpose(2, 1, 0)
    out_t = _build_call(B, V, J, T, C)(eidx, *([vt] * T), jt)
    return out_t.transpose(2, 1, 0)
